# Initial kernel scaffold; baseline (speedup 1.0000x reference)
import jax
import jax.numpy as jnp
from jax.experimental import pallas as pl


def kernel(z, embedding):
    raise NotImplementedError("write your pallas kernel here")


# speedup vs baseline: 1.4947x; 1.4947x over previous
import functools

import jax
import jax.numpy as jnp
from jax import lax
from jax.experimental import pallas as pl
from jax.experimental.pallas import tpu as pltpu
from jax.experimental.pallas import tpu_sc as plsc

DIM = 256
N_CODES = 8192
ROW_BLK = 1024
CODE_BLK = 2048


def _dist_argmin_body(z_ref, e_ref, idx_ref, msum_ref, e2_ref):
    @pl.when(pl.program_id(0) == 0)
    def _():
        e2_ref[...] = jnp.sum(e_ref[...] * e_ref[...], axis=0, keepdims=True)

    zb = z_ref[...]
    run_min = jnp.full((ROW_BLK, 1), jnp.inf, jnp.float32)
    run_idx = jnp.zeros((ROW_BLK, 1), jnp.int32)
    for c in range(N_CODES // CODE_BLK):
        eb = e_ref[:, c * CODE_BLK:(c + 1) * CODE_BLK]
        s = e2_ref[:, c * CODE_BLK:(c + 1) * CODE_BLK] - 2.0 * jnp.dot(
            zb, eb, preferred_element_type=jnp.float32)
        bm = jnp.min(s, axis=1, keepdims=True)
        ii = lax.broadcasted_iota(jnp.int32, s.shape, 1)
        bi = jnp.min(jnp.where(s == bm, ii, N_CODES), axis=1, keepdims=True)
        better = bm < run_min
        run_min = jnp.where(better, bm, run_min)
        run_idx = jnp.where(better, bi + c * CODE_BLK, run_idx)
    z2 = jnp.sum(zb * zb, axis=1, keepdims=True)
    idx_ref[...] = run_idx
    msum_ref[0, 0, 0] = jnp.sum(run_min + z2)


def _dist_argmin(flat_z, embedding):
    n = flat_z.shape[0]
    nb = n // ROW_BLK
    return pl.pallas_call(
        _dist_argmin_body,
        grid=(nb,),
        in_specs=[
            pl.BlockSpec((ROW_BLK, DIM), lambda i: (i, 0)),
            pl.BlockSpec((DIM, N_CODES), lambda i: (0, 0)),
        ],
        out_specs=[
            pl.BlockSpec((ROW_BLK, 1), lambda i: (i, 0)),
            pl.BlockSpec((1, 1, 1), lambda i: (i, 0, 0), memory_space=pltpu.SMEM),
        ],
        out_shape=[
            jax.ShapeDtypeStruct((n, 1), jnp.int32),
            jax.ShapeDtypeStruct((nb, 1, 1), jnp.float32),
        ],
        scratch_shapes=[pltpu.VMEM((1, N_CODES), jnp.float32)],
    )(flat_z, embedding)


@functools.cache
def _make_gather(b_total):
    info = plsc.get_sparse_core_info()
    n_workers = info.num_cores * info.num_subcores
    b_per_w = b_total // n_workers
    chunk = 128
    mesh = plsc.VectorSubcoreMesh(core_axis_name="c", subcore_axis_name="s")

    @functools.partial(
        pl.kernel,
        mesh=mesh,
        out_type=jax.ShapeDtypeStruct((b_total, DIM), jnp.float32),
        scratch_types=[
            pltpu.VMEM((chunk,), jnp.int32),
            pltpu.VMEM((chunk, DIM), jnp.float32),
            pltpu.SemaphoreType.DMA,
        ],
    )
    def gather_kernel(table_hbm, idx_hbm, out_hbm, idx_v, rows_v, sem):
        wid = lax.axis_index("s") * info.num_cores + lax.axis_index("c")
        for j in range(b_per_w // chunk):
            off = wid * b_per_w + j * chunk
            pltpu.sync_copy(idx_hbm.at[pl.ds(off, chunk)], idx_v)
            pltpu.async_copy(table_hbm.at[idx_v], rows_v, sem).wait()
            pltpu.sync_copy(rows_v, out_hbm.at[pl.ds(off, chunk)])

    return gather_kernel


def kernel(z, embedding):
    flat = z.reshape(-1, DIM)
    b_total = flat.shape[0]
    idx2, msums = _dist_argmin(flat, embedding)
    idx = idx2.reshape(b_total)
    table = embedding.T
    quantize = _make_gather(b_total)(table, idx).reshape(z.shape)
    diff = jnp.sum(msums) / (b_total * DIM)
    embed_ind = idx.reshape(z.shape[:-1])
    return (quantize, diff, embed_ind)


# speedup vs baseline: 1.4973x; 1.0018x over previous
import functools

import jax
import jax.numpy as jnp
from jax import lax
from jax.experimental import pallas as pl
from jax.experimental.pallas import tpu as pltpu
from jax.experimental.pallas import tpu_sc as plsc

DIM = 256
N_CODES = 8192
ROW_BLK = 1024
CODE_BLK = 2048


def _dist_argmin_body(z_ref, e_ref, idx_ref, msum_ref, e2_ref):
    @pl.when(pl.program_id(0) == 0)
    def _():
        e2_ref[...] = jnp.sum(e_ref[...] * e_ref[...], axis=0, keepdims=True)

    zb = z_ref[...]
    zb16 = zb.astype(jnp.bfloat16)
    run_min = jnp.full((ROW_BLK, 1), jnp.inf, jnp.float32)
    run_idx = jnp.zeros((ROW_BLK, 1), jnp.int32)
    for c in range(N_CODES // CODE_BLK):
        eb = e_ref[:, c * CODE_BLK:(c + 1) * CODE_BLK].astype(jnp.bfloat16)
        s = e2_ref[:, c * CODE_BLK:(c + 1) * CODE_BLK] - 2.0 * jnp.dot(
            zb16, eb, preferred_element_type=jnp.float32)
        bm = jnp.min(s, axis=1, keepdims=True)
        ii = lax.broadcasted_iota(jnp.int32, s.shape, 1)
        bi = jnp.min(jnp.where(s == bm, ii, N_CODES), axis=1, keepdims=True)
        better = bm < run_min
        run_min = jnp.where(better, bm, run_min)
        run_idx = jnp.where(better, bi + c * CODE_BLK, run_idx)
    z2 = jnp.sum(zb * zb, axis=1, keepdims=True)
    idx_ref[...] = run_idx
    msum_ref[0, 0, 0] = jnp.sum(run_min + z2)


def _dist_argmin(flat_z, embedding):
    n = flat_z.shape[0]
    nb = n // ROW_BLK
    return pl.pallas_call(
        _dist_argmin_body,
        grid=(nb,),
        in_specs=[
            pl.BlockSpec((ROW_BLK, DIM), lambda i: (i, 0)),
            pl.BlockSpec((DIM, N_CODES), lambda i: (0, 0)),
        ],
        out_specs=[
            pl.BlockSpec((ROW_BLK, 1), lambda i: (i, 0)),
            pl.BlockSpec((1, 1, 1), lambda i: (i, 0, 0), memory_space=pltpu.SMEM),
        ],
        out_shape=[
            jax.ShapeDtypeStruct((n, 1), jnp.int32),
            jax.ShapeDtypeStruct((nb, 1, 1), jnp.float32),
        ],
        scratch_shapes=[pltpu.VMEM((1, N_CODES), jnp.float32)],
    )(flat_z, embedding)


@functools.cache
def _make_gather(b_total):
    info = plsc.get_sparse_core_info()
    n_workers = info.num_cores * info.num_subcores
    b_per_w = b_total // n_workers
    chunk = 128
    mesh = plsc.VectorSubcoreMesh(core_axis_name="c", subcore_axis_name="s")

    @functools.partial(
        pl.kernel,
        mesh=mesh,
        out_type=jax.ShapeDtypeStruct((b_total, DIM), jnp.float32),
        scratch_types=[
            pltpu.VMEM((chunk,), jnp.int32),
            pltpu.VMEM((chunk, DIM), jnp.float32),
            pltpu.SemaphoreType.DMA,
        ],
    )
    def gather_kernel(table_hbm, idx_hbm, out_hbm, idx_v, rows_v, sem):
        wid = lax.axis_index("s") * info.num_cores + lax.axis_index("c")
        for j in range(b_per_w // chunk):
            off = wid * b_per_w + j * chunk
            pltpu.sync_copy(idx_hbm.at[pl.ds(off, chunk)], idx_v)
            pltpu.async_copy(table_hbm.at[idx_v], rows_v, sem).wait()
            pltpu.sync_copy(rows_v, out_hbm.at[pl.ds(off, chunk)])

    return gather_kernel


def kernel(z, embedding):
    flat = z.reshape(-1, DIM)
    b_total = flat.shape[0]
    idx2, msums = _dist_argmin(flat, embedding)
    idx = idx2.reshape(b_total)
    table = embedding.T
    quantize = _make_gather(b_total)(table, idx).reshape(z.shape)
    diff = jnp.sum(msums) / (b_total * DIM)
    embed_ind = idx.reshape(z.shape[:-1])
    return (quantize, diff, embed_ind)
